# baseline (speedup 1.0000x reference)
import jax
import jax.numpy as jnp
from jax.experimental import pallas as pl
from jax.experimental.pallas import tpu as pltpu


def _fused_kernel(cond_ref, we_ref, wg_ref, es_ref, er_ref,
                  ce_ref, cg_ref, ge_ref, gg_ref, be_ref, bg_ref,
                  out_ref, w_scr, row_scr):
    n_e, n_g, n_s = 10, 100, 7

    @pl.when(pl.program_id(0) == 0)
    def _init():
        bias = be_ref[0:1, :] + bg_ref[0:1, :]
        w_scr[0:n_e, :] = we_ref[...]
        w_scr[n_e:n_e + n_g, :] = wg_ref[...]
        w_scr[110:117, :] = es_ref[...] + bias
        w_scr[117:128, :] = er_ref[...]
        log2e = 1.4426950408889634
        ge = ge_ref[0, 0]
        gg = gg_ref[0, 0]
        ilane = jax.lax.broadcasted_iota(jnp.int32, (1, 128), 1)
        lane = ilane.astype(jnp.float32)
        is_e = ilane < 10
        is_g = (ilane >= 10) & (ilane < 110)
        is_s = (ilane >= 110) & (ilane < 117)
        ctr_rbf = jnp.concatenate(
            [ce_ref[...], cg_ref[...],
             jnp.zeros((1, 18), jnp.float32)], axis=1)
        ctr = jnp.where(is_e | is_g, ctr_rbf,
                        jnp.where(is_s, lane - 110.0, lane - 117.0))
        row_scr[0:1, :] = ctr
        negg2 = jnp.where(is_e, -log2e * ge,
                          jnp.where(is_g, -log2e * gg, -2e4))
        row_scr[1:2, :] = negg2
        row_scr[2:3, :] = jnp.where(is_e | is_g, 0.0, 1.0)

    c4 = cond_ref[...]
    lane = jax.lax.broadcasted_iota(jnp.int32, (1, 128), 1)
    src = jnp.where(lane < 10, 1,
                    jnp.where(lane < 110, 3,
                              jnp.where(lane < 117, 0, 2)))
    idx = jnp.broadcast_to(src, (c4.shape[0], 128))
    x = jnp.take_along_axis(c4, idx, axis=1)
    d = x - row_scr[0:1, :]
    u = d - row_scr[2:3, :] * (d - jnp.floor(d))
    feats = jnp.exp2(row_scr[1:2, :] * u * u)
    out_ref[...] = jax.lax.dot_general(
        feats, w_scr[...],
        dimension_numbers=(((1,), (0,)), ((), ())),
        preferred_element_type=jnp.float32,
    )


def kernel(condition, centers_eluent, gamma_eluent, W_eluent, b_eluent,
           centers_grain, gamma_grain, W_grain, b_grain,
           emb_silica, emb_replace):
    B = condition.shape[0]
    D = W_eluent.shape[1]

    bB = 2048
    grid = (B // bB,)
    c = lambda i: (0, 0)

    out = pl.pallas_call(
        _fused_kernel,
        grid=grid,
        in_specs=[
            pl.BlockSpec((bB, 4), lambda i: (i, 0)),
            pl.BlockSpec((10, D), c),
            pl.BlockSpec((100, D), c),
            pl.BlockSpec((7, D), c),
            pl.BlockSpec((11, D), c),
            pl.BlockSpec((1, 10), c),
            pl.BlockSpec((1, 100), c),
            pl.BlockSpec((1, 1), c),
            pl.BlockSpec((1, 1), c),
            pl.BlockSpec((1, D), c),
            pl.BlockSpec((1, D), c),
        ],
        out_specs=pl.BlockSpec((bB, D), lambda i: (i, 0)),
        out_shape=jax.ShapeDtypeStruct((B, D), jnp.float32),
        scratch_shapes=[
            pltpu.VMEM((128, 128), jnp.float32),
            pltpu.VMEM((3, 128), jnp.float32),
        ],
    )(condition,
      W_eluent, W_grain, emb_silica, emb_replace,
      centers_eluent.reshape(1, 10), centers_grain.reshape(1, 100),
      gamma_eluent.reshape(1, 1), gamma_grain.reshape(1, 1),
      b_eluent.reshape(1, D), b_grain.reshape(1, D))
    return out


# speedup vs baseline: 1.1620x; 1.1620x over previous
import jax
import jax.numpy as jnp
from jax.experimental import pallas as pl
from jax.experimental.pallas import tpu as pltpu


def _fused_kernel(cond_ref, we_ref, wg_ref, es_ref, er_ref,
                  ce_ref, cg_ref, ge_ref, gg_ref, be_ref, bg_ref,
                  out_ref, w_scr, row_scr):
    n_e, n_g = 10, 100

    @pl.when(pl.program_id(0) == 0)
    def _init():
        bias = be_ref[0:1, :] + bg_ref[0:1, :]
        w_scr[0:n_e, :] = we_ref[...]
        w_scr[n_e:n_e + n_g, :] = wg_ref[...]
        w_scr[110:117, :] = es_ref[...] + bias
        w_scr[117:128, :] = er_ref[...]
        log2e = 1.4426950408889634
        ge = ge_ref[0, 0]
        gg = gg_ref[0, 0]
        ilane = jax.lax.broadcasted_iota(jnp.int32, (1, 128), 1)
        lane = ilane.astype(jnp.float32)
        is_e = ilane < 10
        is_g = (ilane >= 10) & (ilane < 110)
        is_s = (ilane >= 110) & (ilane < 117)
        ctr_rbf = jnp.concatenate(
            [ce_ref[...], cg_ref[...],
             jnp.zeros((1, 18), jnp.float32)], axis=1)
        ctr = jnp.where(is_e | is_g, ctr_rbf,
                        jnp.where(is_s, lane - 110.0, lane - 117.0))
        row_scr[0:1, :] = ctr
        negg2 = jnp.where(is_e, -log2e * ge,
                          jnp.where(is_g, -log2e * gg, -2e4))
        row_scr[1:2, :] = negg2
        row_scr[2:3, :] = jnp.where(is_e | is_g, 0.0, 1.0)

    c4 = cond_ref[...]
    lane = jax.lax.broadcasted_iota(jnp.int32, (1, 128), 1)
    src = jnp.where(lane < 10, 1,
                    jnp.where(lane < 110, 3,
                              jnp.where(lane < 117, 0, 2)))
    idx = jnp.broadcast_to(src, (c4.shape[0], 128))
    x = jnp.take_along_axis(c4, idx, axis=1)
    d = x - row_scr[0:1, :]
    u = d - row_scr[2:3, :] * (d - jnp.floor(d))
    feats = jnp.exp2(row_scr[1:2, :] * u * u)
    out_ref[...] = jax.lax.dot_general(
        feats, w_scr[...],
        dimension_numbers=(((1,), (0,)), ((), ())),
        preferred_element_type=jnp.float32,
    )


def kernel(condition, centers_eluent, gamma_eluent, W_eluent, b_eluent,
           centers_grain, gamma_grain, W_grain, b_grain,
           emb_silica, emb_replace):
    B = condition.shape[0]
    D = W_eluent.shape[1]

    bB = 4096
    grid = (B // bB,)
    c = lambda i: (0, 0)

    out = pl.pallas_call(
        _fused_kernel,
        grid=grid,
        in_specs=[
            pl.BlockSpec((bB, 4), lambda i: (i, 0)),
            pl.BlockSpec((10, D), c),
            pl.BlockSpec((100, D), c),
            pl.BlockSpec((7, D), c),
            pl.BlockSpec((11, D), c),
            pl.BlockSpec((1, 10), c),
            pl.BlockSpec((1, 100), c),
            pl.BlockSpec((1, 1), c),
            pl.BlockSpec((1, 1), c),
            pl.BlockSpec((1, D), c),
            pl.BlockSpec((1, D), c),
        ],
        out_specs=pl.BlockSpec((bB, D), lambda i: (i, 0)),
        out_shape=jax.ShapeDtypeStruct((B, D), jnp.float32),
        scratch_shapes=[
            pltpu.VMEM((128, 128), jnp.float32),
            pltpu.VMEM((3, 128), jnp.float32),
        ],
    )(condition,
      W_eluent, W_grain, emb_silica, emb_replace,
      centers_eluent.reshape(1, 10), centers_grain.reshape(1, 100),
      gamma_eluent.reshape(1, 1), gamma_grain.reshape(1, 1),
      b_eluent.reshape(1, D), b_grain.reshape(1, D))
    return out
